# baseline (speedup 1.0000x reference)
import jax
import jax.numpy as jnp
from jax.experimental import pallas as pl
from jax.experimental.pallas import tpu as pltpu


def _body(cloud_ref, noise_ref, tb_ref, W1_ref, b1_ref, W2_ref, b2_ref,
          vpred_ref, v_ref, cs_ref, mf_ref):
    f32 = jnp.float32
    cloud = cloud_ref[0]
    noise = noise_ref[0]
    tb = tb_ref[0]
    W1 = W1_ref[...]
    b1 = b1_ref[...]
    W2 = W2_ref[...]
    b2 = b2_ref[...]

    N = cloud.shape[0]

    mu = jnp.mean(cloud)
    var = jnp.mean((cloud - mu) ** 2)
    x0 = cloud / jnp.sqrt(var)

    d2 = jnp.zeros((N, N), f32)
    for k in range(3):
        xk = noise[:, k][:, None]
        yk = x0[:, k][None, :]
        d2 = d2 + (xk - yk) ** 2

    eps_s = jnp.float32(0.005 ** 2)
    log2e = jnp.float32(1.4426950408889634)
    ln2 = jnp.float32(0.6931471805599453)
    cs_ref[...] = d2 * (0.5 / eps_s * 1.4426950408889634)
    loga2 = -jnp.log(jnp.float32(N)) * log2e
    logb2 = -jnp.log(jnp.float32(N)) * log2e
    M_g = jnp.zeros((1, N), f32)
    STRIP = 8
    for _ in range(20):
        u = logb2 - M_g

        def fbody(s, _, u=u):
            strip = cs_ref[pl.ds(s * STRIP, STRIP), :]
            A = u - strip
            mx = jnp.max(A, axis=1, keepdims=True)
            sm = jnp.sum(jnp.exp2(A - mx), axis=1, keepdims=True)
            mf_ref[pl.ds(s * STRIP, STRIP), :] = mx + jnp.log2(sm)
            return 0
        jax.lax.fori_loop(0, N // STRIP, fbody, 0)
        M_f = mf_ref[...]

        w = loga2 - M_f
        Ag = w - cs_ref[...]
        mg = jnp.max(Ag, axis=0, keepdims=True)
        M_g = mg + jnp.log2(jnp.sum(jnp.exp2(Ag - mg), axis=0, keepdims=True))
    price = (eps_s * ln2) * M_g

    jidx = jax.lax.broadcasted_iota(jnp.int32, (N, N), 1)
    eps = jnp.float32(1e-3)
    neg_inf = jnp.float32(-jnp.inf)
    mask = None
    for _ in range(5):
        val = d2 + price
        v1 = jnp.max(val, axis=1, keepdims=True)
        best = jnp.min(jnp.where(val == v1, jidx, N), axis=1,
                       keepdims=True)
        mask = jidx == best
        v2 = jnp.max(jnp.where(mask, neg_inf, val), axis=1, keepdims=True)
        bid = v2 - v1 + eps
        scattered = jnp.max(jnp.where(mask, bid, neg_inf), axis=0,
                            keepdims=True)
        price = jnp.where(scattered > neg_inf, scattered, price)

    x0a = jnp.dot(mask.astype(f32), x0, preferred_element_type=f32)

    x_t = (1.0 - tb) * x0a + tb * noise
    feat = jnp.concatenate([x_t, tb], axis=1)
    h = jnp.tanh(jnp.dot(feat, W1, preferred_element_type=f32) + b1)
    vpred_ref[0] = jnp.dot(h, W2, preferred_element_type=f32) + b2
    v_ref[0] = noise - x0a


def kernel(cloud, noise, t, W1, b1, W2, b2):
    B, N, _ = cloud.shape
    H = W1.shape[1]
    tb = jnp.broadcast_to(t[:, None, None], (B, N, 1)).astype(jnp.float32)
    outs = pl.pallas_call(
        _body,
        grid=(B,),
        in_specs=[
            pl.BlockSpec((1, N, 3), lambda b: (b, 0, 0)),
            pl.BlockSpec((1, N, 3), lambda b: (b, 0, 0)),
            pl.BlockSpec((1, N, 1), lambda b: (b, 0, 0)),
            pl.BlockSpec((4, H), lambda b: (0, 0)),
            pl.BlockSpec((1, H), lambda b: (0, 0)),
            pl.BlockSpec((H, 3), lambda b: (0, 0)),
            pl.BlockSpec((1, 3), lambda b: (0, 0)),
        ],
        out_specs=[
            pl.BlockSpec((1, N, 3), lambda b: (b, 0, 0)),
            pl.BlockSpec((1, N, 3), lambda b: (b, 0, 0)),
        ],
        out_shape=[
            jax.ShapeDtypeStruct((B, N, 3), jnp.float32),
            jax.ShapeDtypeStruct((B, N, 3), jnp.float32),
        ],
        scratch_shapes=[
            pltpu.VMEM((N, N), jnp.float32),
            pltpu.VMEM((N, 1), jnp.float32),
        ],
        compiler_params=pltpu.CompilerParams(
            dimension_semantics=("parallel",)),
    )(cloud, noise, tb, W1, b1.reshape(1, H), W2, b2.reshape(1, 3))
    return (outs[0], outs[1])


# speedup vs baseline: 8.2080x; 8.2080x over previous
import jax
import jax.numpy as jnp
from jax.experimental import pallas as pl
from jax.experimental.pallas import tpu as pltpu


def _body(cloud_ref, noise_ref, tb_ref, W1_ref, b1_ref, W2_ref, b2_ref,
          vpred_ref, v_ref):
    f32 = jnp.float32
    cloud = cloud_ref[0]
    noise = noise_ref[0]
    tb = tb_ref[0]
    W1 = W1_ref[...]
    b1 = b1_ref[...]
    W2 = W2_ref[...]
    b2 = b2_ref[...]

    N = cloud.shape[0]

    mu = jnp.mean(cloud)
    var = jnp.mean((cloud - mu) ** 2)
    x0 = cloud / jnp.sqrt(var)

    d2 = jnp.zeros((N, N), f32)
    for k in range(3):
        xk = noise[:, k][:, None]
        yk = x0[:, k][None, :]
        d2 = d2 + (xk - yk) ** 2

    eps_s = jnp.float32(0.005 ** 2)
    log2e = jnp.float32(1.4426950408889634)
    ln2 = jnp.float32(0.6931471805599453)
    Cs = d2 * (0.5 / eps_s * 1.4426950408889634)
    loga2 = -jnp.log(jnp.float32(N)) * log2e
    logb2 = -jnp.log(jnp.float32(N)) * log2e
    M_f = jnp.zeros((N, 1), f32)
    M_g = jnp.zeros((1, N), f32)
    for _ in range(20):
        u = logb2 - M_g
        A = u - Cs
        mx = jnp.max(A, axis=1, keepdims=True)
        M_f = mx + jnp.log2(jnp.sum(jnp.exp2(A - mx), axis=1, keepdims=True))
        w = loga2 - M_f
        Ag = w - Cs
        mg = jnp.max(Ag, axis=0, keepdims=True)
        M_g = mg + jnp.log2(jnp.sum(jnp.exp2(Ag - mg), axis=0, keepdims=True))
    price = (eps_s * ln2) * M_g

    jidx = jax.lax.broadcasted_iota(jnp.int32, (N, N), 1)
    eps = jnp.float32(1e-3)
    neg_inf = jnp.float32(-jnp.inf)
    mask = None
    for r in range(5):
        val = d2 + price
        v1 = jnp.max(val, axis=1, keepdims=True)
        best = jnp.min(jnp.where(val == v1, jidx, N), axis=1,
                       keepdims=True)
        mask = jidx == best
        if r == 4:
            break
        v2 = jnp.max(jnp.where(mask, neg_inf, val), axis=1, keepdims=True)
        bid = v2 - v1 + eps
        scattered = jnp.max(jnp.where(mask, bid, neg_inf), axis=0,
                            keepdims=True)
        price = jnp.where(scattered > neg_inf, scattered, price)

    x0a = jnp.dot(mask.astype(f32), x0, preferred_element_type=f32)

    x_t = (1.0 - tb) * x0a + tb * noise
    feat = jnp.concatenate([x_t, tb], axis=1)
    h = jnp.tanh(jnp.dot(feat, W1, preferred_element_type=f32) + b1)
    vpred_ref[0] = jnp.dot(h, W2, preferred_element_type=f32) + b2
    v_ref[0] = noise - x0a


def kernel(cloud, noise, t, W1, b1, W2, b2):
    B, N, _ = cloud.shape
    H = W1.shape[1]
    tb = jnp.broadcast_to(t[:, None, None], (B, N, 1)).astype(jnp.float32)
    outs = pl.pallas_call(
        _body,
        grid=(B,),
        in_specs=[
            pl.BlockSpec((1, N, 3), lambda b: (b, 0, 0)),
            pl.BlockSpec((1, N, 3), lambda b: (b, 0, 0)),
            pl.BlockSpec((1, N, 1), lambda b: (b, 0, 0)),
            pl.BlockSpec((4, H), lambda b: (0, 0)),
            pl.BlockSpec((1, H), lambda b: (0, 0)),
            pl.BlockSpec((H, 3), lambda b: (0, 0)),
            pl.BlockSpec((1, 3), lambda b: (0, 0)),
        ],
        out_specs=[
            pl.BlockSpec((1, N, 3), lambda b: (b, 0, 0)),
            pl.BlockSpec((1, N, 3), lambda b: (b, 0, 0)),
        ],
        out_shape=[
            jax.ShapeDtypeStruct((B, N, 3), jnp.float32),
            jax.ShapeDtypeStruct((B, N, 3), jnp.float32),
        ],
        compiler_params=pltpu.CompilerParams(
            dimension_semantics=("parallel",)),
    )(cloud, noise, tb, W1, b1.reshape(1, H), W2, b2.reshape(1, 3))
    return (outs[0], outs[1])
